# Initial kernel scaffold; baseline (speedup 1.0000x reference)
import jax
import jax.numpy as jnp
from jax.experimental import pallas as pl


def kernel(y, W_in, W_attr, W_msg, W_upd, W_out):
    raise NotImplementedError("write your pallas kernel here")


# speedup vs baseline: 133.1728x; 133.1728x over previous
import jax
import jax.numpy as jnp
from jax.experimental import pallas as pl

_B = 16
_L = 1000
_H = 50
_LAYERS = 6
_MAX_RADIUS = 0.06
_N_BASIS = 8
_BAR_HALF = 0.05
_T = _L - 2

_BF = jnp.bfloat16


def _silu(x):
    return x * jax.nn.sigmoid(x)


def _dot(a, b):
    return jax.lax.dot_general(
        a, b, (((1,), (0,)), ((), ())),
        preferred_element_type=jnp.float32)


def _gnn_kernel(rb_ref, wmT_ref, wuT_ref, woT_ref, h0_ref, out_ref):
    f32 = jnp.float32
    rb14 = rb_ref[0, 0:8].astype(_BF)
    rb43 = rb_ref[0, 8:16].astype(_BF)
    rb45 = rb_ref[0, 16:24].astype(_BF)
    one = jnp.ones((1, _T), _BF)
    zer = jnp.zeros((1, _T), _BF)
    zer50 = jnp.zeros((_H, 128), _BF)

    h1 = jnp.broadcast_to(h0_ref[0], (_H, 128))
    h4 = jnp.broadcast_to(h0_ref[0], (_H, _T))
    h3 = jnp.broadcast_to(h0_ref[1], (_H, _T))
    h5 = jnp.broadcast_to(h0_ref[1], (_H, _T))
    for l in range(_LAYERS):
        wm = wmT_ref[l]
        wu = wuT_ref[l]
        h1b = h1.astype(_BF)
        h4b = h4.astype(_BF)
        h3b = h3.astype(_BF)
        h5b = h5.astype(_BF)
        h1c = jnp.broadcast_to(h1b[:, 0:1], (_H, _T))
        mD = _silu(_dot(wm, jnp.concatenate([h1c, rb14, one, zer], 0)))
        mB3 = _silu(_dot(wm, jnp.concatenate([h4b, rb43, zer, one], 0)))
        mB5 = _silu(_dot(wm, jnp.concatenate([h4b, rb45, zer, one], 0)))
        h1 = h1 + _silu(_dot(wu, jnp.concatenate([h1b, zer50], 0)))
        h4 = h4 + _silu(_dot(wu, jnp.concatenate([h4b, mD.astype(_BF)], 0)))
        h3 = h3 + _silu(_dot(wu, jnp.concatenate([h3b, mB3.astype(_BF)], 0)))
        h5 = h5 + _silu(_dot(wu, jnp.concatenate([h5b, mB5.astype(_BF)], 0)))

    wo = woT_ref[:]
    o = (_dot(wo, h3.astype(_BF)) + _dot(wo, h4.astype(_BF))) \
        + _dot(wo, h5.astype(_BF))
    out_ref[0] = jnp.zeros((3, _L), f32)
    out_ref[0, 0:2, 1:_L - 1] = o


def kernel(y, W_in, W_attr, W_msg, W_upd, W_out):
    f32 = jnp.float32
    y = y.astype(f32)
    bf32 = lambda a: a.astype(_BF).astype(f32)

    d1 = y[:, 1:-1, 0:2] - y[:, :-2, 0:2]
    d2 = y[:, 2:, 0:2] - y[:, 1:-1, 0:2]
    l1 = jnp.linalg.norm(d1, axis=-1)
    phi1 = jnp.arctan2(d1[..., 1], d1[..., 0])
    phi2 = jnp.arctan2(d2[..., 1], d2[..., 0])
    theta = phi2 - phi1
    gamma2 = y[:, 1:-1, 2] - 0.5 * (phi1 + phi2)
    ang = gamma2 + 0.5 * theta
    dvx = -jnp.sin(ang)
    dvy = jnp.cos(ang)
    tx = (l1 + _BAR_HALF * dvx) - l1
    ty = _BAR_HALF * dvy
    bx = (l1 - _BAR_HALF * dvx) - l1
    by = jnp.zeros_like(ty) - _BAR_HALF * dvy
    r14 = jnp.sqrt(l1 * l1)
    r43 = jnp.sqrt(tx * tx + ty * ty)
    r45 = jnp.sqrt(bx * bx + by * by)

    def basis(r):
        rc = jnp.linspace(0.0, _MAX_RADIUS, _N_BASIS)
        rb = jnp.exp(-jnp.square((r[..., None] - rc[None, :])
                                 / (_MAX_RADIUS / _N_BASIS)))
        cut = 0.5 * (jnp.cos(jnp.pi * jnp.clip(r / _MAX_RADIUS, 0.0, 1.0))
                     + 1.0)
        return rb * cut[..., None]

    rb_all = jnp.concatenate(
        [jnp.transpose(basis(r), (0, 2, 1)) for r in (r14, r43, r45)],
        axis=1)

    wmT = jnp.transpose(W_msg, (0, 2, 1)).astype(_BF)
    wuT = jnp.transpose(W_upd, (0, 2, 1)).astype(_BF)
    woT = W_out[:, 0:2].T.astype(_BF)
    h_c = bf32(W_in[0]) + bf32(W_attr[0])
    h_t = bf32(W_in[0]) + bf32(W_attr[1])
    h0 = jnp.stack([h_c, h_t])[:, :, None]

    full = lambda *s: pl.BlockSpec(s, lambda b: (0,) * len(s))
    out_t = pl.pallas_call(
        _gnn_kernel,
        grid=(_B,),
        in_specs=[
            pl.BlockSpec((1, 24, _T), lambda b: (b, 0, 0)),
            full(_LAYERS, _H, _H + _N_BASIS + 2),
            full(_LAYERS, _H, 2 * _H),
            full(2, _H),
            full(2, _H, 1),
        ],
        out_specs=pl.BlockSpec((1, 3, _L), lambda b: (b, 0, 0)),
        out_shape=jax.ShapeDtypeStruct((_B, 3, _L), f32),
    )(rb_all, wmT, wuT, woT, h0)

    return jnp.transpose(out_t, (0, 2, 1))


# speedup vs baseline: 136.2159x; 1.0229x over previous
import jax
import jax.numpy as jnp
from jax.experimental import pallas as pl

_B = 16
_L = 1000
_H = 50
_LAYERS = 6
_MAX_RADIUS = 0.06
_N_BASIS = 8
_BAR_HALF = 0.05
_T = _L - 2

_BF = jnp.bfloat16


def _silu(x):
    return x * jax.nn.sigmoid(x)


def _dot(a, b):
    return jax.lax.dot_general(
        a, b, (((1,), (0,)), ((), ())),
        preferred_element_type=jnp.float32)


def _gnn_kernel(rb_ref, wmT_ref, wuT_ref, woT_ref, h0_ref, out_ref):
    f32 = jnp.float32
    rb14 = rb_ref[0, 0:8]
    rb43 = rb_ref[0, 8:16]
    rb45 = rb_ref[0, 16:24]
    one = jnp.ones((1, _T), f32)
    zer = jnp.zeros((1, _T), f32)
    zer50 = jnp.zeros((_H, 128), f32)

    h1 = jnp.broadcast_to(h0_ref[0], (_H, 128))
    h4 = jnp.broadcast_to(h0_ref[0], (_H, _T))
    h3 = jnp.broadcast_to(h0_ref[1], (_H, _T))
    h5 = jnp.broadcast_to(h0_ref[1], (_H, _T))
    for l in range(_LAYERS):
        wm = wmT_ref[l]
        wu = wuT_ref[l]
        h1c = jnp.broadcast_to(h1[:, 0:1], (_H, _T))
        mD = _silu(_dot(wm, jnp.concatenate([h1c, rb14, one, zer], 0)))
        mB3 = _silu(_dot(wm, jnp.concatenate([h4, rb43, zer, one], 0)))
        mB5 = _silu(_dot(wm, jnp.concatenate([h4, rb45, zer, one], 0)))
        h1 = h1 + _silu(_dot(wu, jnp.concatenate([h1, zer50], 0)))
        h4 = h4 + _silu(_dot(wu, jnp.concatenate([h4, mD], 0)))
        h3 = h3 + _silu(_dot(wu, jnp.concatenate([h3, mB3], 0)))
        h5 = h5 + _silu(_dot(wu, jnp.concatenate([h5, mB5], 0)))

    wo = woT_ref[:]
    o = (_dot(wo, h3) + _dot(wo, h4)) + _dot(wo, h5)
    out_ref[0] = jnp.zeros((3, _L), f32)
    out_ref[0, 0:2, 1:_L - 1] = o


def kernel(y, W_in, W_attr, W_msg, W_upd, W_out):
    f32 = jnp.float32
    y = y.astype(f32)
    bf32 = lambda a: a.astype(_BF).astype(f32)

    d1 = y[:, 1:-1, 0:2] - y[:, :-2, 0:2]
    d2 = y[:, 2:, 0:2] - y[:, 1:-1, 0:2]
    l1 = jnp.linalg.norm(d1, axis=-1)
    phi1 = jnp.arctan2(d1[..., 1], d1[..., 0])
    phi2 = jnp.arctan2(d2[..., 1], d2[..., 0])
    theta = phi2 - phi1
    gamma2 = y[:, 1:-1, 2] - 0.5 * (phi1 + phi2)
    ang = gamma2 + 0.5 * theta
    dvx = -jnp.sin(ang)
    dvy = jnp.cos(ang)
    tx = (l1 + _BAR_HALF * dvx) - l1
    ty = _BAR_HALF * dvy
    bx = (l1 - _BAR_HALF * dvx) - l1
    by = jnp.zeros_like(ty) - _BAR_HALF * dvy
    r14 = jnp.sqrt(l1 * l1)
    r43 = jnp.sqrt(tx * tx + ty * ty)
    r45 = jnp.sqrt(bx * bx + by * by)

    def basis(r):
        rc = jnp.linspace(0.0, _MAX_RADIUS, _N_BASIS)
        rb = jnp.exp(-jnp.square((r[..., None] - rc[None, :])
                                 / (_MAX_RADIUS / _N_BASIS)))
        cut = 0.5 * (jnp.cos(jnp.pi * jnp.clip(r / _MAX_RADIUS, 0.0, 1.0))
                     + 1.0)
        return rb * cut[..., None]

    rb_all = jnp.concatenate(
        [jnp.transpose(basis(r), (0, 2, 1)) for r in (r14, r43, r45)],
        axis=1)

    wmT = jnp.transpose(W_msg, (0, 2, 1))
    wuT = jnp.transpose(W_upd, (0, 2, 1))
    woT = W_out[:, 0:2].T
    h_c = bf32(W_in[0]) + bf32(W_attr[0])
    h_t = bf32(W_in[0]) + bf32(W_attr[1])
    h0 = jnp.stack([h_c, h_t])[:, :, None]

    full = lambda *s: pl.BlockSpec(s, lambda b: (0,) * len(s))
    out_t = pl.pallas_call(
        _gnn_kernel,
        grid=(_B,),
        in_specs=[
            pl.BlockSpec((1, 24, _T), lambda b: (b, 0, 0)),
            full(_LAYERS, _H, _H + _N_BASIS + 2),
            full(_LAYERS, _H, 2 * _H),
            full(2, _H),
            full(2, _H, 1),
        ],
        out_specs=pl.BlockSpec((1, 3, _L), lambda b: (b, 0, 0)),
        out_shape=jax.ShapeDtypeStruct((_B, 3, _L), f32),
    )(rb_all, wmT, wuT, woT, h0)

    return jnp.transpose(out_t, (0, 2, 1))


# speedup vs baseline: 144.4236x; 1.0603x over previous
import jax
import jax.numpy as jnp
from jax.experimental import pallas as pl

_B = 16
_L = 1000
_H = 50
_LAYERS = 6
_MAX_RADIUS = 0.06
_N_BASIS = 8
_BAR_HALF = 0.05
_T = _L - 2

_BF = jnp.bfloat16
_GB = 4
_TT = _GB * _T


def _silu(x):
    return x * jax.nn.sigmoid(x)


def _dot(a, b):
    return jax.lax.dot_general(
        a, b, (((1,), (0,)), ((), ())),
        preferred_element_type=jnp.float32)


def _gnn_kernel(rb_ref, wmT_ref, wuT_ref, woT_ref, h0_ref, out_ref):
    f32 = jnp.float32
    rb14 = rb_ref[0, 0:8]
    rb43 = rb_ref[0, 8:16]
    rb45 = rb_ref[0, 16:24]
    one = jnp.ones((1, _TT), f32)
    zer = jnp.zeros((1, _TT), f32)
    zer50 = jnp.zeros((_H, 128), f32)

    h1 = jnp.broadcast_to(h0_ref[0], (_H, 128))
    h4 = jnp.broadcast_to(h0_ref[0], (_H, _TT))
    h3 = jnp.broadcast_to(h0_ref[1], (_H, _TT))
    h5 = jnp.broadcast_to(h0_ref[1], (_H, _TT))
    for l in range(_LAYERS):
        wm = wmT_ref[l]
        wu = wuT_ref[l]
        h1c = jnp.broadcast_to(h1[:, 0:1], (_H, _TT))
        mD = _silu(_dot(wm, jnp.concatenate([h1c, rb14, one, zer], 0)))
        mB3 = _silu(_dot(wm, jnp.concatenate([h4, rb43, zer, one], 0)))
        mB5 = _silu(_dot(wm, jnp.concatenate([h4, rb45, zer, one], 0)))
        h1 = h1 + _silu(_dot(wu, jnp.concatenate([h1, zer50], 0)))
        h4 = h4 + _silu(_dot(wu, jnp.concatenate([h4, mD], 0)))
        h3 = h3 + _silu(_dot(wu, jnp.concatenate([h3, mB3], 0)))
        h5 = h5 + _silu(_dot(wu, jnp.concatenate([h5, mB5], 0)))

    wo = woT_ref[:]
    o = (_dot(wo, h3) + _dot(wo, h4)) + _dot(wo, h5)
    out_ref[0] = jnp.zeros((3, _GB * _L), f32)
    for i in range(_GB):
        out_ref[0, 0:2, i * _L + 1:(i + 1) * _L - 1] = o[:, i * _T:(i + 1) * _T]


def kernel(y, W_in, W_attr, W_msg, W_upd, W_out):
    f32 = jnp.float32
    y = y.astype(f32)
    bf32 = lambda a: a.astype(_BF).astype(f32)

    d1 = y[:, 1:-1, 0:2] - y[:, :-2, 0:2]
    d2 = y[:, 2:, 0:2] - y[:, 1:-1, 0:2]
    l1 = jnp.linalg.norm(d1, axis=-1)
    phi1 = jnp.arctan2(d1[..., 1], d1[..., 0])
    phi2 = jnp.arctan2(d2[..., 1], d2[..., 0])
    theta = phi2 - phi1
    gamma2 = y[:, 1:-1, 2] - 0.5 * (phi1 + phi2)
    ang = gamma2 + 0.5 * theta
    dvx = -jnp.sin(ang)
    dvy = jnp.cos(ang)
    tx = (l1 + _BAR_HALF * dvx) - l1
    ty = _BAR_HALF * dvy
    bx = (l1 - _BAR_HALF * dvx) - l1
    by = jnp.zeros_like(ty) - _BAR_HALF * dvy
    r14 = jnp.sqrt(l1 * l1)
    r43 = jnp.sqrt(tx * tx + ty * ty)
    r45 = jnp.sqrt(bx * bx + by * by)

    def basis(r):
        rc = jnp.linspace(0.0, _MAX_RADIUS, _N_BASIS)
        rb = jnp.exp(-jnp.square((r[..., None] - rc[None, :])
                                 / (_MAX_RADIUS / _N_BASIS)))
        cut = 0.5 * (jnp.cos(jnp.pi * jnp.clip(r / _MAX_RADIUS, 0.0, 1.0))
                     + 1.0)
        return rb * cut[..., None]

    rb_all = jnp.concatenate(
        [jnp.transpose(basis(r), (0, 2, 1)) for r in (r14, r43, r45)],
        axis=1)
    rb_all = jnp.transpose(rb_all.reshape(_B // _GB, _GB, 24, _T),
                           (0, 2, 1, 3)).reshape(_B // _GB, 24, _TT)

    wmT = jnp.transpose(W_msg, (0, 2, 1))
    wuT = jnp.transpose(W_upd, (0, 2, 1))
    woT = W_out[:, 0:2].T
    h_c = bf32(W_in[0]) + bf32(W_attr[0])
    h_t = bf32(W_in[0]) + bf32(W_attr[1])
    h0 = jnp.stack([h_c, h_t])[:, :, None]

    full = lambda *s: pl.BlockSpec(s, lambda b: (0,) * len(s))
    out_t = pl.pallas_call(
        _gnn_kernel,
        grid=(_B // _GB,),
        in_specs=[
            pl.BlockSpec((1, 24, _TT), lambda b: (b, 0, 0)),
            full(_LAYERS, _H, _H + _N_BASIS + 2),
            full(_LAYERS, _H, 2 * _H),
            full(2, _H),
            full(2, _H, 1),
        ],
        out_specs=pl.BlockSpec((1, 3, _GB * _L), lambda b: (b, 0, 0)),
        out_shape=jax.ShapeDtypeStruct((_B // _GB, 3, _GB * _L), f32),
    )(rb_all, wmT, wuT, woT, h0)

    out_t = out_t.reshape(_B // _GB, 3, _GB, _L)
    return jnp.transpose(out_t, (0, 2, 3, 1)).reshape(_B, _L, 3)


# speedup vs baseline: 152.7488x; 1.0576x over previous
import jax
import jax.numpy as jnp
from jax.experimental import pallas as pl

_B = 16
_L = 1000
_H = 50
_LAYERS = 6
_MAX_RADIUS = 0.06
_N_BASIS = 8
_BAR_HALF = 0.05
_T = _L - 2

_BF = jnp.bfloat16
_GB = 8
_TT = _GB * _T


def _silu(x):
    return x * jax.nn.sigmoid(x)


def _dot(a, b):
    return jax.lax.dot_general(
        a, b, (((1,), (0,)), ((), ())),
        preferred_element_type=jnp.float32)


def _gnn_kernel(rb_ref, wmT_ref, wuT_ref, woT_ref, h0_ref, out_ref):
    f32 = jnp.float32
    rb14 = rb_ref[0, 0:8]
    rb43 = rb_ref[0, 8:16]
    rb45 = rb_ref[0, 16:24]
    one = jnp.ones((1, _TT), f32)
    zer = jnp.zeros((1, _TT), f32)
    zer50 = jnp.zeros((_H, 128), f32)

    h1 = jnp.broadcast_to(h0_ref[0], (_H, 128))
    h4 = jnp.broadcast_to(h0_ref[0], (_H, _TT))
    h3 = jnp.broadcast_to(h0_ref[1], (_H, _TT))
    h5 = jnp.broadcast_to(h0_ref[1], (_H, _TT))
    for l in range(_LAYERS):
        wm = wmT_ref[l]
        wu = wuT_ref[l]
        h1c = jnp.broadcast_to(h1[:, 0:1], (_H, _TT))
        mD = _silu(_dot(wm, jnp.concatenate([h1c, rb14, one, zer], 0)))
        mB3 = _silu(_dot(wm, jnp.concatenate([h4, rb43, zer, one], 0)))
        mB5 = _silu(_dot(wm, jnp.concatenate([h4, rb45, zer, one], 0)))
        h1 = h1 + _silu(_dot(wu, jnp.concatenate([h1, zer50], 0)))
        h4 = h4 + _silu(_dot(wu, jnp.concatenate([h4, mD], 0)))
        h3 = h3 + _silu(_dot(wu, jnp.concatenate([h3, mB3], 0)))
        h5 = h5 + _silu(_dot(wu, jnp.concatenate([h5, mB5], 0)))

    wo = woT_ref[:]
    o = (_dot(wo, h3) + _dot(wo, h4)) + _dot(wo, h5)
    out_ref[0] = jnp.zeros((3, _GB * _L), f32)
    for i in range(_GB):
        out_ref[0, 0:2, i * _L + 1:(i + 1) * _L - 1] = o[:, i * _T:(i + 1) * _T]


def kernel(y, W_in, W_attr, W_msg, W_upd, W_out):
    f32 = jnp.float32
    y = y.astype(f32)
    bf32 = lambda a: a.astype(_BF).astype(f32)

    d1 = y[:, 1:-1, 0:2] - y[:, :-2, 0:2]
    d2 = y[:, 2:, 0:2] - y[:, 1:-1, 0:2]
    l1 = jnp.linalg.norm(d1, axis=-1)
    phi1 = jnp.arctan2(d1[..., 1], d1[..., 0])
    phi2 = jnp.arctan2(d2[..., 1], d2[..., 0])
    theta = phi2 - phi1
    gamma2 = y[:, 1:-1, 2] - 0.5 * (phi1 + phi2)
    ang = gamma2 + 0.5 * theta
    dvx = -jnp.sin(ang)
    dvy = jnp.cos(ang)
    tx = (l1 + _BAR_HALF * dvx) - l1
    ty = _BAR_HALF * dvy
    bx = (l1 - _BAR_HALF * dvx) - l1
    by = jnp.zeros_like(ty) - _BAR_HALF * dvy
    r14 = jnp.sqrt(l1 * l1)
    r43 = jnp.sqrt(tx * tx + ty * ty)
    r45 = jnp.sqrt(bx * bx + by * by)

    def basis(r):
        rc = jnp.linspace(0.0, _MAX_RADIUS, _N_BASIS)
        rb = jnp.exp(-jnp.square((r[..., None] - rc[None, :])
                                 / (_MAX_RADIUS / _N_BASIS)))
        cut = 0.5 * (jnp.cos(jnp.pi * jnp.clip(r / _MAX_RADIUS, 0.0, 1.0))
                     + 1.0)
        return rb * cut[..., None]

    rb_all = jnp.concatenate(
        [jnp.transpose(basis(r), (0, 2, 1)) for r in (r14, r43, r45)],
        axis=1)
    rb_all = jnp.transpose(rb_all.reshape(_B // _GB, _GB, 24, _T),
                           (0, 2, 1, 3)).reshape(_B // _GB, 24, _TT)

    wmT = jnp.transpose(W_msg, (0, 2, 1))
    wuT = jnp.transpose(W_upd, (0, 2, 1))
    woT = W_out[:, 0:2].T
    h_c = bf32(W_in[0]) + bf32(W_attr[0])
    h_t = bf32(W_in[0]) + bf32(W_attr[1])
    h0 = jnp.stack([h_c, h_t])[:, :, None]

    full = lambda *s: pl.BlockSpec(s, lambda b: (0,) * len(s))
    out_t = pl.pallas_call(
        _gnn_kernel,
        grid=(_B // _GB,),
        in_specs=[
            pl.BlockSpec((1, 24, _TT), lambda b: (b, 0, 0)),
            full(_LAYERS, _H, _H + _N_BASIS + 2),
            full(_LAYERS, _H, 2 * _H),
            full(2, _H),
            full(2, _H, 1),
        ],
        out_specs=pl.BlockSpec((1, 3, _GB * _L), lambda b: (b, 0, 0)),
        out_shape=jax.ShapeDtypeStruct((_B // _GB, 3, _GB * _L), f32),
    )(rb_all, wmT, wuT, woT, h0)

    out_t = out_t.reshape(_B // _GB, 3, _GB, _L)
    return jnp.transpose(out_t, (0, 2, 3, 1)).reshape(_B, _L, 3)


# speedup vs baseline: 154.0859x; 1.0088x over previous
import jax
import jax.numpy as jnp
from jax.experimental import pallas as pl

_B = 16
_L = 1000
_H = 50
_LAYERS = 6
_MAX_RADIUS = 0.06
_N_BASIS = 8
_BAR_HALF = 0.05
_T = _L - 2

_BF = jnp.bfloat16
_GB = 16
_TT = _GB * _T


def _silu(x):
    return x * jax.nn.sigmoid(x)


def _dot(a, b):
    return jax.lax.dot_general(
        a, b, (((1,), (0,)), ((), ())),
        preferred_element_type=jnp.float32)


def _gnn_kernel(rb_ref, wmT_ref, wuT_ref, woT_ref, h0_ref, out_ref):
    f32 = jnp.float32
    rb14 = rb_ref[0, 0:8]
    rb43 = rb_ref[0, 8:16]
    rb45 = rb_ref[0, 16:24]
    one = jnp.ones((1, _TT), f32)
    zer = jnp.zeros((1, _TT), f32)
    zer50 = jnp.zeros((_H, 128), f32)

    h1 = jnp.broadcast_to(h0_ref[0], (_H, 128))
    h4 = jnp.broadcast_to(h0_ref[0], (_H, _TT))
    h3 = jnp.broadcast_to(h0_ref[1], (_H, _TT))
    h5 = jnp.broadcast_to(h0_ref[1], (_H, _TT))
    for l in range(_LAYERS):
        wm = wmT_ref[l]
        wu = wuT_ref[l]
        h1c = jnp.broadcast_to(h1[:, 0:1], (_H, _TT))
        mD = _silu(_dot(wm, jnp.concatenate([h1c, rb14, one, zer], 0)))
        mB3 = _silu(_dot(wm, jnp.concatenate([h4, rb43, zer, one], 0)))
        mB5 = _silu(_dot(wm, jnp.concatenate([h4, rb45, zer, one], 0)))
        h1 = h1 + _silu(_dot(wu, jnp.concatenate([h1, zer50], 0)))
        h4 = h4 + _silu(_dot(wu, jnp.concatenate([h4, mD], 0)))
        h3 = h3 + _silu(_dot(wu, jnp.concatenate([h3, mB3], 0)))
        h5 = h5 + _silu(_dot(wu, jnp.concatenate([h5, mB5], 0)))

    wo = woT_ref[:]
    o = (_dot(wo, h3) + _dot(wo, h4)) + _dot(wo, h5)
    out_ref[0] = jnp.zeros((3, _GB * _L), f32)
    for i in range(_GB):
        out_ref[0, 0:2, i * _L + 1:(i + 1) * _L - 1] = o[:, i * _T:(i + 1) * _T]


def kernel(y, W_in, W_attr, W_msg, W_upd, W_out):
    f32 = jnp.float32
    y = y.astype(f32)
    bf32 = lambda a: a.astype(_BF).astype(f32)

    d1 = y[:, 1:-1, 0:2] - y[:, :-2, 0:2]
    d2 = y[:, 2:, 0:2] - y[:, 1:-1, 0:2]
    l1 = jnp.linalg.norm(d1, axis=-1)
    phi1 = jnp.arctan2(d1[..., 1], d1[..., 0])
    phi2 = jnp.arctan2(d2[..., 1], d2[..., 0])
    theta = phi2 - phi1
    gamma2 = y[:, 1:-1, 2] - 0.5 * (phi1 + phi2)
    ang = gamma2 + 0.5 * theta
    dvx = -jnp.sin(ang)
    dvy = jnp.cos(ang)
    tx = (l1 + _BAR_HALF * dvx) - l1
    ty = _BAR_HALF * dvy
    bx = (l1 - _BAR_HALF * dvx) - l1
    by = jnp.zeros_like(ty) - _BAR_HALF * dvy
    r14 = jnp.sqrt(l1 * l1)
    r43 = jnp.sqrt(tx * tx + ty * ty)
    r45 = jnp.sqrt(bx * bx + by * by)

    def basis(r):
        rc = jnp.linspace(0.0, _MAX_RADIUS, _N_BASIS)
        rb = jnp.exp(-jnp.square((r[..., None] - rc[None, :])
                                 / (_MAX_RADIUS / _N_BASIS)))
        cut = 0.5 * (jnp.cos(jnp.pi * jnp.clip(r / _MAX_RADIUS, 0.0, 1.0))
                     + 1.0)
        return rb * cut[..., None]

    rb_all = jnp.concatenate(
        [jnp.transpose(basis(r), (0, 2, 1)) for r in (r14, r43, r45)],
        axis=1)
    rb_all = jnp.transpose(rb_all.reshape(_B // _GB, _GB, 24, _T),
                           (0, 2, 1, 3)).reshape(_B // _GB, 24, _TT)

    wmT = jnp.transpose(W_msg, (0, 2, 1))
    wuT = jnp.transpose(W_upd, (0, 2, 1))
    woT = W_out[:, 0:2].T
    h_c = bf32(W_in[0]) + bf32(W_attr[0])
    h_t = bf32(W_in[0]) + bf32(W_attr[1])
    h0 = jnp.stack([h_c, h_t])[:, :, None]

    full = lambda *s: pl.BlockSpec(s, lambda b: (0,) * len(s))
    out_t = pl.pallas_call(
        _gnn_kernel,
        grid=(_B // _GB,),
        in_specs=[
            pl.BlockSpec((1, 24, _TT), lambda b: (b, 0, 0)),
            full(_LAYERS, _H, _H + _N_BASIS + 2),
            full(_LAYERS, _H, 2 * _H),
            full(2, _H),
            full(2, _H, 1),
        ],
        out_specs=pl.BlockSpec((1, 3, _GB * _L), lambda b: (b, 0, 0)),
        out_shape=jax.ShapeDtypeStruct((_B // _GB, 3, _GB * _L), f32),
    )(rb_all, wmT, wuT, woT, h0)

    out_t = out_t.reshape(_B // _GB, 3, _GB, _L)
    return jnp.transpose(out_t, (0, 2, 3, 1)).reshape(_B, _L, 3)
